# baseline (speedup 1.0000x reference)
import functools

import jax
import jax.numpy as jnp
from jax.experimental import pallas as pl
from jax.experimental.pallas import tpu as pltpu
from jax.experimental.pallas import tpu_sc as plsc

NQ = 4
KC = 1024
D = 384
TM = 512
NTOK = 16 * 576
BETA = 0.25

NH = NTOK // 2
NTH = NH // TM

NW = 32
BPW = NH // NW
GCH = 72


def _dist_step(r_ref, q_ref, cb_ref, nrm_ref, rout_ref, idx_ref, ss_ref,
               *, first):
    E = cb_ref[0]
    n = nrm_ref[0]
    if first:
        r = r_ref[...]
        rout_ref[...] = r
    else:
        r = r_ref[...] - q_ref[...]
        rout_ref[...] = r
    part = jnp.full((8, 128), jnp.sum(r * r), jnp.float32)

    @pl.when(pl.program_id(0) == 0)
    def _():
        ss_ref[...] = part

    @pl.when(pl.program_id(0) != 0)
    def _():
        ss_ref[...] += part

    a = jnp.sum(r * r, axis=1, keepdims=True)
    dot = jax.lax.dot_general(
        r, E, (((1,), (1,)), ((), ())),
        precision=jax.lax.Precision.DEFAULT,
        preferred_element_type=jnp.float32,
    )
    dist = (a - 2.0 * dot) + n
    m = jnp.min(dist, axis=1, keepdims=True)
    iota = jax.lax.broadcasted_iota(jnp.int32, dist.shape, 1)
    idx_ref[...] = jnp.min(jnp.where(dist == m, iota, KC), axis=1,
                           keepdims=True)


def _dist_call(r, q, cb_s, nrm_s, first):
    return pl.pallas_call(
        functools.partial(_dist_step, first=first),
        grid=(NTH,),
        in_specs=[
            pl.BlockSpec((TM, D), lambda t: (t, 0)),
            pl.BlockSpec((TM, D), lambda t: (t, 0)),
            pl.BlockSpec((1, KC, D), lambda t: (0, 0, 0)),
            pl.BlockSpec((1, 1, KC), lambda t: (0, 0, 0)),
        ],
        out_specs=[
            pl.BlockSpec((TM, D), lambda t: (t, 0)),
            pl.BlockSpec((TM, 1), lambda t: (t, 0)),
            pl.BlockSpec((8, 128), lambda t: (0, 0)),
        ],
        out_shape=[
            jax.ShapeDtypeStruct((NH, D), jnp.float32),
            jax.ShapeDtypeStruct((NH, 1), jnp.int32),
            jax.ShapeDtypeStruct((8, 128), jnp.float32),
        ],
    )(r, q, cb_s, nrm_s)


def _final_step(x_ref, r_ref, q_ref, out_ref, ss_ref):
    r_new = r_ref[...] - q_ref[...]
    out_ref[...] = x_ref[...] - r_new
    part = jnp.full((8, 128), jnp.sum(r_new * r_new), jnp.float32)

    @pl.when(pl.program_id(0) == 0)
    def _():
        ss_ref[...] = part

    @pl.when(pl.program_id(0) != 0)
    def _():
        ss_ref[...] += part


def _final_call(xh, r, q):
    return pl.pallas_call(
        _final_step,
        grid=(NTH,),
        in_specs=[
            pl.BlockSpec((TM, D), lambda t: (t, 0)),
            pl.BlockSpec((TM, D), lambda t: (t, 0)),
            pl.BlockSpec((TM, D), lambda t: (t, 0)),
        ],
        out_specs=[
            pl.BlockSpec((TM, D), lambda t: (t, 0)),
            pl.BlockSpec((8, 128), lambda t: (0, 0)),
        ],
        out_shape=[
            jax.ShapeDtypeStruct((NH, D), jnp.float32),
            jax.ShapeDtypeStruct((8, 128), jnp.float32),
        ],
    )(xh, r, q)


def _sc_gather(table, idx):
    mesh = plsc.VectorSubcoreMesh(core_axis_name="c", subcore_axis_name="s")

    @functools.partial(
        pl.kernel, mesh=mesh,
        out_type=jax.ShapeDtypeStruct((NH, D), jnp.float32),
        scratch_types=[
            pltpu.VMEM((BPW,), jnp.int32),
            pltpu.VMEM((BPW, D), jnp.float32),
            pltpu.SemaphoreType.DMA,
        ],
    )
    def k(table_hbm, idx_hbm, out_hbm, idx_v, rows_v, sem):
        wid = jax.lax.axis_index("s") * 2 + jax.lax.axis_index("c")
        base = wid * BPW
        pltpu.sync_copy(idx_hbm.at[pl.ds(base, BPW)], idx_v)
        cps = [
            pltpu.async_copy(
                table_hbm.at[idx_v.at[pl.ds(j * GCH, GCH)]],
                rows_v.at[pl.ds(j * GCH, GCH)],
                sem,
            )
            for j in range(BPW // GCH)
        ]
        for cp in cps:
            cp.wait()
        pltpu.sync_copy(rows_v, out_hbm.at[pl.ds(base, BPW)])

    return k(table, idx)


def kernel(x, codebooks):
    xf = x.reshape(NTOK, D)
    nrm = jnp.sum(codebooks ** 2, axis=2)[:, None, :]

    xh = [xf[:NH], xf[NH:]]
    zeros = jnp.zeros((NH, D), jnp.float32)
    r = [xh[0], xh[1]]
    qprev = [zeros, zeros]
    idxs = [[], []]
    sss = [[], []]
    for s in range(NQ):
        for h in (0, 1):
            r[h], idx_s, ss_prev = _dist_call(
                r[h], qprev[h], codebooks[s:s + 1], nrm[s:s + 1],
                first=(s == 0))
            idxs[h].append(idx_s)
            sss[h].append(ss_prev)
            qprev[h] = _sc_gather(codebooks[s], idx_s.reshape(NH))

    outs, ss_last = [], []
    for h in (0, 1):
        o, sl = _final_call(xh[h], r[h], qprev[h])
        outs.append(o)
        ss_last.append(sl)

    quantized_out = jnp.concatenate(outs, axis=0).reshape(x.shape)
    all_indices = jnp.concatenate(
        [jnp.concatenate(idxs[0], axis=1), jnp.concatenate(idxs[1], axis=1)],
        axis=0).reshape(x.shape[0], x.shape[1], NQ)
    ss_stage = [sss[0][s + 1][0, 0] + sss[1][s + 1][0, 0] for s in range(NQ - 1)]
    ss_stage.append(ss_last[0][0, 0] + ss_last[1][0, 0])
    vql = (jnp.stack(ss_stage) / jnp.float32(NTOK * D)).reshape(1, NQ)
    qql = BETA * vql
    return quantized_out, all_indices, vql, qql


# speedup vs baseline: 1.2055x; 1.2055x over previous
import functools

import jax
import jax.numpy as jnp
from jax.experimental import pallas as pl
from jax.experimental.pallas import tpu as pltpu
from jax.experimental.pallas import tpu_sc as plsc

NQ = 4
KC = 1024
D = 384
TM = 1024
NTOK = 16 * 576
BETA = 0.25

NH = NTOK
NTH = NH // TM

NW = 32
BPW = NH // NW
GCH = 96


def _dist_step(r_ref, q_ref, cb_ref, nrm_ref, rout_ref, idx_ref, ss_ref,
               *, first):
    E = cb_ref[0]
    n = nrm_ref[0]
    if first:
        r = r_ref[...]
        rout_ref[...] = r
    else:
        r = r_ref[...] - q_ref[...]
        rout_ref[...] = r
    part = jnp.full((8, 128), jnp.sum(r * r), jnp.float32)

    @pl.when(pl.program_id(0) == 0)
    def _():
        ss_ref[...] = part

    @pl.when(pl.program_id(0) != 0)
    def _():
        ss_ref[...] += part

    a = jnp.sum(r * r, axis=1, keepdims=True)
    dot = jax.lax.dot_general(
        r, E, (((1,), (1,)), ((), ())),
        precision=jax.lax.Precision.DEFAULT,
        preferred_element_type=jnp.float32,
    )
    dist = (a - 2.0 * dot) + n
    m = jnp.min(dist, axis=1, keepdims=True)
    iota = jax.lax.broadcasted_iota(jnp.int32, dist.shape, 1)
    idx_ref[...] = jnp.min(jnp.where(dist == m, iota, KC), axis=1,
                           keepdims=True)


def _dist_call(r, q, cb_s, nrm_s, first):
    return pl.pallas_call(
        functools.partial(_dist_step, first=first),
        grid=(NTH,),
        in_specs=[
            pl.BlockSpec((TM, D), lambda t: (t, 0)),
            pl.BlockSpec((TM, D), lambda t: (t, 0)),
            pl.BlockSpec((1, KC, D), lambda t: (0, 0, 0)),
            pl.BlockSpec((1, 1, KC), lambda t: (0, 0, 0)),
        ],
        out_specs=[
            pl.BlockSpec((TM, D), lambda t: (t, 0)),
            pl.BlockSpec((TM, 1), lambda t: (t, 0)),
            pl.BlockSpec((8, 128), lambda t: (0, 0)),
        ],
        out_shape=[
            jax.ShapeDtypeStruct((NH, D), jnp.float32),
            jax.ShapeDtypeStruct((NH, 1), jnp.int32),
            jax.ShapeDtypeStruct((8, 128), jnp.float32),
        ],
    )(r, q, cb_s, nrm_s)


def _final_step(x_ref, r_ref, q_ref, out_ref, ss_ref):
    r_new = r_ref[...] - q_ref[...]
    out_ref[...] = x_ref[...] - r_new
    part = jnp.full((8, 128), jnp.sum(r_new * r_new), jnp.float32)

    @pl.when(pl.program_id(0) == 0)
    def _():
        ss_ref[...] = part

    @pl.when(pl.program_id(0) != 0)
    def _():
        ss_ref[...] += part


def _final_call(xh, r, q):
    return pl.pallas_call(
        _final_step,
        grid=(NTH,),
        in_specs=[
            pl.BlockSpec((TM, D), lambda t: (t, 0)),
            pl.BlockSpec((TM, D), lambda t: (t, 0)),
            pl.BlockSpec((TM, D), lambda t: (t, 0)),
        ],
        out_specs=[
            pl.BlockSpec((TM, D), lambda t: (t, 0)),
            pl.BlockSpec((8, 128), lambda t: (0, 0)),
        ],
        out_shape=[
            jax.ShapeDtypeStruct((NH, D), jnp.float32),
            jax.ShapeDtypeStruct((8, 128), jnp.float32),
        ],
    )(xh, r, q)


def _sc_gather(table, idx):
    mesh = plsc.VectorSubcoreMesh(core_axis_name="c", subcore_axis_name="s")

    @functools.partial(
        pl.kernel, mesh=mesh,
        out_type=jax.ShapeDtypeStruct((NH, D), jnp.float32),
        scratch_types=[
            pltpu.VMEM((BPW,), jnp.int32),
            pltpu.VMEM((BPW, D), jnp.float32),
            pltpu.SemaphoreType.DMA,
        ],
    )
    def k(table_hbm, idx_hbm, out_hbm, idx_v, rows_v, sem):
        wid = jax.lax.axis_index("s") * 2 + jax.lax.axis_index("c")
        base = wid * BPW
        pltpu.sync_copy(idx_hbm.at[pl.ds(base, BPW)], idx_v)
        cps = [
            pltpu.async_copy(
                table_hbm.at[idx_v.at[pl.ds(j * GCH, GCH)]],
                rows_v.at[pl.ds(j * GCH, GCH)],
                sem,
            )
            for j in range(BPW // GCH)
        ]
        for cp in cps:
            cp.wait()
        pltpu.sync_copy(rows_v, out_hbm.at[pl.ds(base, BPW)])

    return k(table, idx)


def kernel(x, codebooks):
    xf = x.reshape(NTOK, D)
    nrm = jnp.sum(codebooks ** 2, axis=2)[:, None, :]

    zeros = jnp.zeros((NH, D), jnp.float32)
    r = xf
    qprev = zeros
    idxs = []
    sss = []
    for s in range(NQ):
        r, idx_s, ss_prev = _dist_call(
            r, qprev, codebooks[s:s + 1], nrm[s:s + 1], first=(s == 0))
        idxs.append(idx_s)
        sss.append(ss_prev)
        qprev = _sc_gather(codebooks[s], idx_s.reshape(NH))

    out, ss_last = _final_call(xf, r, qprev)
    sss.append(ss_last)

    quantized_out = out.reshape(x.shape)
    all_indices = jnp.concatenate(idxs, axis=1).reshape(
        x.shape[0], x.shape[1], NQ)
    ss_stage = [sss[s + 1][0, 0] for s in range(NQ)]
    vql = (jnp.stack(ss_stage) / jnp.float32(NTOK * D)).reshape(1, NQ)
    qql = BETA * vql
    return quantized_out, all_indices, vql, qql
